# baseline (speedup 1.0000x reference)
import functools

import jax
import jax.numpy as jnp
from jax import lax
from jax.experimental import pallas as pl
from jax.experimental.pallas import tpu as pltpu
from jax.experimental.pallas import tpu_sc as plsc

T = 256
BN = 1024


def _sc_gather_rows(table, idx, C, nbuf):
    V, D = table.shape
    dtype = table.dtype
    R = idx.shape[0]
    info = plsc.get_sparse_core_info()
    NC, NS = info.num_cores, info.num_subcores
    NW = NC * NS
    rpw = R // NW
    nch = rpw // C
    nbuf = min(nbuf, nch)
    mesh = plsc.VectorSubcoreMesh(core_axis_name="c", subcore_axis_name="s")

    @functools.partial(
        pl.kernel, mesh=mesh,
        out_type=jax.ShapeDtypeStruct((R, D), dtype),
        scratch_types=(
            [pltpu.VMEM((rpw,), jnp.int32)]
            + [pltpu.VMEM((C, D), dtype) for _ in range(nbuf)]
            + [pltpu.SemaphoreType.DMA for _ in range(2 * nbuf)]
        ),
    )
    def k(table_hbm, idx_hbm, out_hbm, *scr):
        idx_v = scr[0]
        bufs = scr[1:1 + nbuf]
        gsem = scr[1 + nbuf:1 + 2 * nbuf]
        osem = scr[1 + 2 * nbuf:1 + 3 * nbuf]
        wid = lax.axis_index("s") * NC + lax.axis_index("c")
        base = wid * rpw
        pltpu.sync_copy(idx_hbm.at[pl.ds(base, rpw)], idx_v)

        def gather(c, p):
            return pltpu.async_copy(
                table_hbm.at[idx_v.at[pl.ds(c * C, C)]], bufs[p], gsem[p])

        gathers = [gather(i, i) for i in range(nbuf)]
        outs = [None] * nbuf
        for c in range(nch):
            p = c % nbuf
            if c > 0 and (c - 1) + nbuf < nch:
                q = (c - 1) % nbuf
                outs[q].wait()
                gathers[q] = gather(c - 1 + nbuf, q)
            gathers[p].wait()
            outs[p] = pltpu.async_copy(
                bufs[p], out_hbm.at[pl.ds(base + c * C, C)], osem[p])
        for c in range(max(0, nch - nbuf), nch):
            outs[c % nbuf].wait()

    return k(table, idx)


def _sc_dispatch_pack(points, idx):
    V, D = points.shape
    R = idx.shape[0]
    info = plsc.get_sparse_core_info()
    NC, NS = info.num_cores, info.num_subcores
    NW = NC * NS
    rpw = R // NW
    C = 16
    nch = rpw // C
    mesh = plsc.VectorSubcoreMesh(core_axis_name="c", subcore_axis_name="s")

    @functools.partial(
        pl.kernel, mesh=mesh,
        out_type=jax.ShapeDtypeStruct((R, D // 2), jnp.int32),
        scratch_types=(
            [pltpu.VMEM((rpw,), jnp.int32)]
            + [pltpu.VMEM((C, D), jnp.float32) for _ in range(2)]
            + [pltpu.VMEM((C, D // 2), jnp.int32) for _ in range(2)]
            + [pltpu.SemaphoreType.DMA for _ in range(4)]
        ),
    )
    def k(tab_hbm, idx_hbm, out_hbm, idx_v, in0, in1, pk0, pk1,
          g0, g1, o0, o1):
        ins = (in0, in1)
        pks = (pk0, pk1)
        gsem = (g0, g1)
        osem = (o0, o1)
        wid = lax.axis_index("s") * NC + lax.axis_index("c")
        base = wid * rpw
        pltpu.sync_copy(idx_hbm.at[pl.ds(base, rpw)], idx_v)

        def gather(c, p):
            return pltpu.async_copy(
                tab_hbm.at[idx_v.at[pl.ds(c * C, C)]], ins[p], gsem[p])

        def pack_chunk(p):
            src_ref = ins[p]
            dst_ref = pks[p]

            def row(r, _):
                for kk in range(D // 2 // 16):
                    lo = jax.lax.bitcast_convert_type(
                        src_ref[r, pl.ds(16 * kk, 16)], jnp.int32)
                    hi = jax.lax.bitcast_convert_type(
                        src_ref[r, pl.ds(D // 2 + 16 * kk, 16)], jnp.int32)
                    lor = lo + 32767 + jnp.bitwise_and(
                        jnp.right_shift(lo, 16), 1)
                    hir = hi + 32767 + jnp.bitwise_and(
                        jnp.right_shift(hi, 16), 1)
                    word = jnp.bitwise_or(
                        jnp.bitwise_and(jnp.right_shift(lor, 16), 0xFFFF),
                        jnp.bitwise_and(hir, jnp.int32(-65536)))
                    dst_ref[r, pl.ds(16 * kk, 16)] = word
                return 0

            lax.fori_loop(0, C, row, 0)

        gathers = [gather(0, 0), None]
        outs = [None, None]
        for c in range(nch):
            p = c % 2
            q = 1 - p
            if c + 1 < nch:
                if outs[q] is not None:
                    outs[q].wait()
                gathers[q] = gather(c + 1, q)
            gathers[p].wait()
            pack_chunk(p)
            outs[p] = pltpu.async_copy(
                pks[p], out_hbm.at[pl.ds(base + c * C, C)], osem[p])
        outs[(nch - 1) % 2].wait()
        if nch > 1:
            outs[nch % 2].wait()

    return k(points, idx)


def _gmm_body(eot_ref, tc_ref, er_ref, nr_ref,
              x_ref, w_hbm, b_ref, o_ref,
              wb0, wb1, sem0, sem1, st_ref):
    NT = pl.num_programs(0)
    n = pl.program_id(0)
    m = pl.program_id(1)
    mt = pl.num_programs(1) - 1
    nruns = nr_ref[0]
    s = n * nruns + tc_ref[m]
    wbufs = (wb0, wb1)
    sems = (sem0, sem1)

    def w_copy(seq, slot):
        r = seq - (seq // nruns) * nruns
        e = er_ref[r]
        nn = seq // nruns
        return pltpu.make_async_copy(
            w_hbm.at[e, :, pl.ds(nn * BN, BN)], wbufs[slot], sems[slot])

    @pl.when(jnp.logical_and(n == 0, m == 0))
    def _():
        w_copy(0, 0).start()
        st_ref[0] = -1
        st_ref[1] = 0

    for par in (0, 1):
        @pl.when(jnp.logical_and(st_ref[0] != s, s % 2 == par))
        def _(par=par):
            w_copy(s, par).wait()
            st_ref[0] = s

        @pl.when(jnp.logical_and(
            jnp.logical_and(st_ref[1] < s + 1, s + 1 < NT * nruns),
            (s + 1) % 2 == par))
        def _(par=par):
            w_copy(s + 1, par).start()
            st_ref[1] = s + 1

    xw = x_ref[...]
    K = xw.shape[1]
    lo = jax.lax.bitcast_convert_type(lax.shift_left(xw, 16), jnp.float32)
    hi = jax.lax.bitcast_convert_type(
        jnp.bitwise_and(xw, jnp.int32(-65536)), jnp.float32)

    for par in (0, 1):
        @pl.when(jnp.logical_and(m != mt, s % 2 == par))
        def _(wb=wbufs[par]):
            acc = jnp.dot(lo, wb[:K, :], preferred_element_type=jnp.float32)
            acc = acc + jnp.dot(hi, wb[K:, :],
                                preferred_element_type=jnp.float32)
            o_ref[...] = acc + b_ref[0]

    @pl.when(m == mt)
    def _():
        o_ref[...] = jnp.zeros_like(o_ref)


def _grouped_matmul(x_packed, W, b, eot, tc, er, nruns, P, D, E):
    MT = P // T
    NT = D // BN
    grid_spec = pltpu.PrefetchScalarGridSpec(
        num_scalar_prefetch=4,
        grid=(NT, MT + 1),
        in_specs=[
            pl.BlockSpec((T, D // 2),
                         lambda n, m, *pref: (jnp.minimum(m, MT - 1), 0)),
            pl.BlockSpec(memory_space=pl.ANY),
            pl.BlockSpec((1, 1, BN),
                         lambda n, m, eot, *pref: (eot[m], 0, n)),
        ],
        out_specs=pl.BlockSpec((T, BN), lambda n, m, *pref: (m, n)),
        scratch_shapes=[
            pltpu.VMEM((D, BN), jnp.float32),
            pltpu.VMEM((D, BN), jnp.float32),
            pltpu.SemaphoreType.DMA,
            pltpu.SemaphoreType.DMA,
            pltpu.SMEM((2,), jnp.int32),
        ],
    )
    return pl.pallas_call(
        _gmm_body,
        grid_spec=grid_spec,
        out_shape=jax.ShapeDtypeStruct((P + T, D), jnp.float32),
    )(eot, tc, er, nruns, x_packed, W, b.reshape(E, 1, D))


def _route(e, E, P, MT):
    N = e.shape[0]
    order = jnp.argsort(e, stable=True).astype(jnp.int32)
    oh = (e[:, None] == jnp.arange(E, dtype=jnp.int32)[None, :]).astype(jnp.int32)
    counts = jnp.sum(oh, axis=0)
    rank = jnp.sum((jnp.cumsum(oh, axis=0) - 1) * oh, axis=1)
    padded = ((counts + T - 1) // T) * T
    goff = jnp.cumsum(padded) - padded
    coff = jnp.cumsum(counts) - counts
    e_c = jnp.minimum(e, E - 1)
    dest = jnp.where(e < E, goff[e_c] + rank, P).astype(jnp.int32)
    j = jnp.arange(P, dtype=jnp.int32)
    e_slot = jnp.clip(
        jnp.sum(j[:, None] >= goff[None, :], axis=1) - 1, 0, E - 1)
    pos = coff[e_slot] + j - goff[e_slot]
    src = order[jnp.clip(pos, 0, N - 1)]
    tile_start = goff // T
    eot = (jnp.sum(jnp.arange(MT + 1, dtype=jnp.int32)[:, None]
                   >= tile_start[None, :], axis=1) - 1)
    eot = jnp.clip(eot, 0, E - 1).astype(jnp.int32)
    return src, dest, eot


def kernel(points, region_ids, W, b):
    N, D = points.shape
    E = W.shape[0]
    P = ((N + E * (T - 1) + T - 1) // T) * T
    MT = P // T
    e = region_ids.astype(jnp.int32)
    src, dest, eot = _route(e, E, P, MT)
    x_packed = _sc_dispatch_pack(points, src)
    changes = jnp.concatenate(
        [jnp.zeros((1,), jnp.int32), (eot[1:] != eot[:-1]).astype(jnp.int32)])
    tc = jnp.cumsum(changes).astype(jnp.int32)
    nruns = (tc[-1] + 1).reshape(1).astype(jnp.int32)
    er = jnp.zeros((MT + 1,), jnp.int32).at[tc].set(eot)
    y_ext = _grouped_matmul(x_packed, W, b, eot, tc, er, nruns, P, D, E)
    out = _sc_gather_rows(y_ext, dest, C=16, nbuf=3)
    return out.reshape(-1, D)


# speedup vs baseline: 1.0597x; 1.0597x over previous
import functools

import jax
import jax.numpy as jnp
from jax import lax
from jax.experimental import pallas as pl
from jax.experimental.pallas import tpu as pltpu
from jax.experimental.pallas import tpu_sc as plsc

T = 256
BN = 2048


def _sc_gather_rows(table, idx, C, nbuf):
    V, D = table.shape
    dtype = table.dtype
    R = idx.shape[0]
    info = plsc.get_sparse_core_info()
    NC, NS = info.num_cores, info.num_subcores
    NW = NC * NS
    rpw = R // NW
    nch = rpw // C
    nbuf = min(nbuf, nch)
    mesh = plsc.VectorSubcoreMesh(core_axis_name="c", subcore_axis_name="s")

    @functools.partial(
        pl.kernel, mesh=mesh,
        out_type=jax.ShapeDtypeStruct((R, D), dtype),
        scratch_types=(
            [pltpu.VMEM((rpw,), jnp.int32)]
            + [pltpu.VMEM((C, D), dtype) for _ in range(nbuf)]
            + [pltpu.SemaphoreType.DMA for _ in range(2 * nbuf)]
        ),
    )
    def k(table_hbm, idx_hbm, out_hbm, *scr):
        idx_v = scr[0]
        bufs = scr[1:1 + nbuf]
        gsem = scr[1 + nbuf:1 + 2 * nbuf]
        osem = scr[1 + 2 * nbuf:1 + 3 * nbuf]
        wid = lax.axis_index("s") * NC + lax.axis_index("c")
        base = wid * rpw
        pltpu.sync_copy(idx_hbm.at[pl.ds(base, rpw)], idx_v)

        def gather(c, p):
            return pltpu.async_copy(
                table_hbm.at[idx_v.at[pl.ds(c * C, C)]], bufs[p], gsem[p])

        gathers = [gather(i, i) for i in range(nbuf)]
        outs = [None] * nbuf
        for c in range(nch):
            p = c % nbuf
            if c > 0 and (c - 1) + nbuf < nch:
                q = (c - 1) % nbuf
                outs[q].wait()
                gathers[q] = gather(c - 1 + nbuf, q)
            gathers[p].wait()
            outs[p] = pltpu.async_copy(
                bufs[p], out_hbm.at[pl.ds(base + c * C, C)], osem[p])
        for c in range(max(0, nch - nbuf), nch):
            outs[c % nbuf].wait()

    return k(table, idx)


def _sc_dispatch_pack(points, idx):
    V, D = points.shape
    R = idx.shape[0]
    info = plsc.get_sparse_core_info()
    NC, NS = info.num_cores, info.num_subcores
    NW = NC * NS
    rpw = R // NW
    C = 16
    nch = rpw // C
    mesh = plsc.VectorSubcoreMesh(core_axis_name="c", subcore_axis_name="s")

    @functools.partial(
        pl.kernel, mesh=mesh,
        out_type=jax.ShapeDtypeStruct((R, D // 2), jnp.int32),
        scratch_types=(
            [pltpu.VMEM((rpw,), jnp.int32)]
            + [pltpu.VMEM((C, D), jnp.float32) for _ in range(2)]
            + [pltpu.VMEM((C, D // 2), jnp.int32) for _ in range(2)]
            + [pltpu.SemaphoreType.DMA for _ in range(4)]
        ),
    )
    def k(tab_hbm, idx_hbm, out_hbm, idx_v, in0, in1, pk0, pk1,
          g0, g1, o0, o1):
        ins = (in0, in1)
        pks = (pk0, pk1)
        gsem = (g0, g1)
        osem = (o0, o1)
        wid = lax.axis_index("s") * NC + lax.axis_index("c")
        base = wid * rpw
        pltpu.sync_copy(idx_hbm.at[pl.ds(base, rpw)], idx_v)

        def gather(c, p):
            return pltpu.async_copy(
                tab_hbm.at[idx_v.at[pl.ds(c * C, C)]], ins[p], gsem[p])

        def pack_chunk(p):
            src_ref = ins[p]
            dst_ref = pks[p]

            def row(r, _):
                for kk in range(D // 2 // 16):
                    lo = jax.lax.bitcast_convert_type(
                        src_ref[r, pl.ds(16 * kk, 16)], jnp.int32)
                    hi = jax.lax.bitcast_convert_type(
                        src_ref[r, pl.ds(D // 2 + 16 * kk, 16)], jnp.int32)
                    word = jnp.bitwise_or(
                        jnp.bitwise_and(
                            jnp.right_shift(lo + 32768, 16), 0xFFFF),
                        jnp.bitwise_and(hi + 32768, jnp.int32(-65536)))
                    dst_ref[r, pl.ds(16 * kk, 16)] = word
                return 0

            lax.fori_loop(0, C, row, 0)

        gathers = [gather(0, 0), None]
        outs = [None, None]
        for c in range(nch):
            p = c % 2
            q = 1 - p
            if c + 1 < nch:
                if outs[q] is not None:
                    outs[q].wait()
                gathers[q] = gather(c + 1, q)
            gathers[p].wait()
            pack_chunk(p)
            outs[p] = pltpu.async_copy(
                pks[p], out_hbm.at[pl.ds(base + c * C, C)], osem[p])
        outs[(nch - 1) % 2].wait()
        if nch > 1:
            outs[nch % 2].wait()

    return k(points, idx)


def _gmm_body(eot_ref, tc_ref, er_ref, nr_ref,
              x_ref, w_hbm, b_ref, o_ref,
              wb0, wb1, sem0, sem1, st_ref):
    NT = pl.num_programs(0)
    n = pl.program_id(0)
    m = pl.program_id(1)
    mt = pl.num_programs(1) - 1
    nruns = nr_ref[0]
    s = n * nruns + tc_ref[m]
    wbufs = (wb0, wb1)
    sems = (sem0, sem1)

    def w_copy(seq, slot):
        r = seq - (seq // nruns) * nruns
        e = er_ref[r]
        nn = seq // nruns
        return pltpu.make_async_copy(
            w_hbm.at[e, :, pl.ds(nn * BN, BN)], wbufs[slot], sems[slot])

    @pl.when(jnp.logical_and(n == 0, m == 0))
    def _():
        w_copy(0, 0).start()
        st_ref[0] = -1
        st_ref[1] = 0

    for par in (0, 1):
        @pl.when(jnp.logical_and(st_ref[0] != s, s % 2 == par))
        def _(par=par):
            w_copy(s, par).wait()
            st_ref[0] = s

        @pl.when(jnp.logical_and(
            jnp.logical_and(st_ref[1] < s + 1, s + 1 < NT * nruns),
            (s + 1) % 2 == par))
        def _(par=par):
            w_copy(s + 1, par).start()
            st_ref[1] = s + 1

    xw = x_ref[...]
    K = xw.shape[1]
    lo = jax.lax.bitcast_convert_type(lax.shift_left(xw, 16), jnp.float32)
    hi = jax.lax.bitcast_convert_type(
        jnp.bitwise_and(xw, jnp.int32(-65536)), jnp.float32)

    for par in (0, 1):
        @pl.when(jnp.logical_and(m != mt, s % 2 == par))
        def _(wb=wbufs[par]):
            acc = jnp.dot(lo, wb[:K, :], preferred_element_type=jnp.float32)
            acc = acc + jnp.dot(hi, wb[K:, :],
                                preferred_element_type=jnp.float32)
            o_ref[...] = acc + b_ref[0]

    @pl.when(m == mt)
    def _():
        o_ref[...] = jnp.zeros_like(o_ref)


def _grouped_matmul(x_packed, W, b, eot, tc, er, nruns, P, D, E):
    MT = P // T
    NT = D // BN
    grid_spec = pltpu.PrefetchScalarGridSpec(
        num_scalar_prefetch=4,
        grid=(NT, MT + 1),
        in_specs=[
            pl.BlockSpec((T, D // 2),
                         lambda n, m, *pref: (jnp.minimum(m, MT - 1), 0)),
            pl.BlockSpec(memory_space=pl.ANY),
            pl.BlockSpec((1, 1, BN),
                         lambda n, m, eot, *pref: (eot[m], 0, n)),
        ],
        out_specs=pl.BlockSpec((T, BN), lambda n, m, *pref: (m, n)),
        scratch_shapes=[
            pltpu.VMEM((D, BN), jnp.float32),
            pltpu.VMEM((D, BN), jnp.float32),
            pltpu.SemaphoreType.DMA,
            pltpu.SemaphoreType.DMA,
            pltpu.SMEM((2,), jnp.int32),
        ],
    )
    return pl.pallas_call(
        _gmm_body,
        grid_spec=grid_spec,
        out_shape=jax.ShapeDtypeStruct((P + T, D), jnp.float32),
    )(eot, tc, er, nruns, x_packed, W, b.reshape(E, 1, D))


def _route(e, E, P, MT):
    N = e.shape[0]
    order = jnp.argsort(e, stable=True).astype(jnp.int32)
    oh = (e[:, None] == jnp.arange(E, dtype=jnp.int32)[None, :]).astype(jnp.int32)
    counts = jnp.sum(oh, axis=0)
    rank = jnp.sum((jnp.cumsum(oh, axis=0) - 1) * oh, axis=1)
    padded = ((counts + T - 1) // T) * T
    goff = jnp.cumsum(padded) - padded
    coff = jnp.cumsum(counts) - counts
    e_c = jnp.minimum(e, E - 1)
    dest = jnp.where(e < E, goff[e_c] + rank, P).astype(jnp.int32)
    j = jnp.arange(P, dtype=jnp.int32)
    e_slot = jnp.clip(
        jnp.sum(j[:, None] >= goff[None, :], axis=1) - 1, 0, E - 1)
    pos = coff[e_slot] + j - goff[e_slot]
    src = order[jnp.clip(pos, 0, N - 1)]
    tile_start = goff // T
    eot = (jnp.sum(jnp.arange(MT + 1, dtype=jnp.int32)[:, None]
                   >= tile_start[None, :], axis=1) - 1)
    eot = jnp.clip(eot, 0, E - 1).astype(jnp.int32)
    return src, dest, eot, counts, goff


def kernel(points, region_ids, W, b):
    N, D = points.shape
    E = W.shape[0]
    P = ((N + E * (T - 1) + T - 1) // T) * T
    MT = P // T
    e = region_ids.astype(jnp.int32)
    src, dest, eot, counts, goff = _route(e, E, P, MT)
    x_packed = _sc_dispatch_pack(points, src)
    changes = jnp.concatenate(
        [jnp.zeros((1,), jnp.int32), (eot[1:] != eot[:-1]).astype(jnp.int32)])
    tc = jnp.cumsum(changes).astype(jnp.int32)
    nruns = (tc[-1] + 1).reshape(1).astype(jnp.int32)
    er = jnp.zeros((MT + 1,), jnp.int32).at[tc].set(eot)
    y_ext = _grouped_matmul(x_packed, W, b, eot, tc, er, nruns, P, D, E)
    out = _sc_gather_rows(y_ext, dest, C=16, nbuf=3)
    return out.reshape(-1, D)
